# Initial kernel scaffold; baseline (speedup 1.0000x reference)
import jax
import jax.numpy as jnp
from jax.experimental import pallas as pl


def kernel(x, edge_index, batch_index, weight, w_ih, w_hh, b_ih, b_hh, w_head, b_head):
    raise NotImplementedError("write your pallas kernel here")


# speedup vs baseline: 2.3821x; 2.3821x over previous
import functools

import jax
import jax.numpy as jnp
from jax import lax
from jax.experimental import pallas as pl
from jax.experimental.pallas import tpu as pltpu
from jax.experimental.pallas import tpu_sc as plsc

N = 10000
E = 320000
H = 128
G = 64
L = 3

NC, NS = 2, 16
NW = NC * NS
CHUNK = 128
EPW = 10240
CPW = EPW // CHUNK
EPAD = EPW * NW
NPAD = 10112
ZR = NPAD // NS

BN = 1000
NB = N // BN

_PREC = lax.Precision.HIGHEST

_sc_mesh = plsc.VectorSubcoreMesh(core_axis_name="c", subcore_axis_name="s")


@functools.partial(
    pl.kernel,
    out_type=jax.ShapeDtypeStruct((NC, NPAD, H), jnp.float32),
    mesh=_sc_mesh,
    scratch_types=[
        pltpu.VMEM((2, CHUNK), jnp.int32),
        pltpu.VMEM((CHUNK, H), jnp.float32),
        pltpu.VMEM_SHARED((NPAD, H), jnp.float32),
        pltpu.SemaphoreType.DMA,
    ],
)
def _sc_edge_aggregate(m_hbm, epack_hbm, zeros_hbm, out_hbm, ev, rows, agg_sh, sem):
    c = lax.axis_index("c")
    s = lax.axis_index("s")
    pltpu.sync_copy(zeros_hbm.at[pl.ds(s * ZR, ZR)], agg_sh.at[pl.ds(s * ZR, ZR)])
    plsc.subcore_barrier()
    wid = c * NS + s

    def body(g, carry):
        gch = wid * CPW + g
        pltpu.sync_copy(epack_hbm.at[gch], ev)
        pltpu.async_copy(m_hbm.at[ev.at[0]], rows, sem).wait()
        pltpu.sync_copy(rows, agg_sh.at[ev.at[1]], add=True)
        return carry

    lax.fori_loop(0, CPW, body, 0)
    plsc.subcore_barrier()
    pltpu.sync_copy(agg_sh.at[pl.ds(s * ZR, ZR)],
                    out_hbm.at[c, pl.ds(s * ZR, ZR)])


def _sigmoid(v):
    return 1.0 / (1.0 + jnp.exp(-v))


def _mm_body(x_ref, w_ref, o_ref):
    o_ref[...] = jnp.dot(x_ref[...], w_ref[...], precision=_PREC,
                         preferred_element_type=jnp.float32)


def _matmul(x, w):
    return pl.pallas_call(
        _mm_body,
        grid=(NB,),
        in_specs=[pl.BlockSpec((BN, H), lambda j: (j, 0)),
                  pl.BlockSpec((H, H), lambda j: (0, 0))],
        out_specs=pl.BlockSpec((BN, H), lambda j: (j, 0)),
        out_shape=jax.ShapeDtypeStruct((N, H), jnp.float32),
    )(x, w)


def _gru_math(p0, p1, h, wih, whh, bih, bhh):
    agg = p0[...] + p1[...]
    gi = jnp.dot(agg, wih[...], precision=_PREC,
                 preferred_element_type=jnp.float32) + bih[...]
    gh = jnp.dot(h[...], whh[...], precision=_PREC,
                 preferred_element_type=jnp.float32) + bhh[...]
    hv = h[...]
    r = _sigmoid(gi[:, 0:H] + gh[:, 0:H])
    z = _sigmoid(gi[:, H:2 * H] + gh[:, H:2 * H])
    n = jnp.tanh(gi[:, 2 * H:3 * H] + r * gh[:, 2 * H:3 * H])
    return (1.0 - z) * n + z * hv


def _gru_body(p0, p1, h, wih, whh, bih, bhh, wn, ho, mo):
    hn = _gru_math(p0, p1, h, wih, whh, bih, bhh)
    ho[...] = hn
    mo[...] = jnp.dot(hn, wn[...], precision=_PREC,
                      preferred_element_type=jnp.float32)


def _gru_next(p0, p1, h, wihT, whhT, bih2, bhh2, wnext):
    full = lambda r, c: pl.BlockSpec((r, c), lambda j: (0, 0))
    blk = pl.BlockSpec((BN, H), lambda j: (j, 0))
    return pl.pallas_call(
        _gru_body,
        grid=(NB,),
        in_specs=[blk, blk, blk, full(H, 3 * H), full(H, 3 * H),
                  full(1, 3 * H), full(1, 3 * H), full(H, H)],
        out_specs=[blk, blk],
        out_shape=[jax.ShapeDtypeStruct((N, H), jnp.float32),
                   jax.ShapeDtypeStruct((N, H), jnp.float32)],
    )(p0, p1, h, wihT, whhT, bih2, bhh2, wnext)


def _final_body(p0, p1, h, x, wih, whh, bih, bhh, wxr, whr, bh, bidx, out):
    j = pl.program_id(0)
    hn = _gru_math(p0, p1, h, wih, whh, bih, bhh)
    lg2 = x[...] * wxr[...] + hn * whr[...]
    rowlog = jnp.sum(lg2, axis=1, keepdims=True) + bh[...]
    bi = bidx[0]
    onehot = (lax.broadcasted_iota(jnp.int32, (G, BN), 0) == bi
              ).astype(jnp.float32)
    pp = jnp.dot(onehot, rowlog, precision=_PREC,
                 preferred_element_type=jnp.float32)

    @pl.when(j == 0)
    def _init():
        out[...] = jnp.zeros_like(out)

    out[...] += pp

    @pl.when(j == NB - 1)
    def _fin():
        out[...] = _sigmoid(out[...])


def _final(p0, p1, h, x, wihT, whhT, bih2, bhh2, wxr, whr, bh2, bidx3):
    full = lambda r, c: pl.BlockSpec((r, c), lambda j: (0, 0))
    blk = pl.BlockSpec((BN, H), lambda j: (j, 0))
    return pl.pallas_call(
        _final_body,
        grid=(NB,),
        in_specs=[blk, blk, blk, blk, full(H, 3 * H), full(H, 3 * H),
                  full(1, 3 * H), full(1, 3 * H), full(1, H), full(1, H),
                  full(1, 1), pl.BlockSpec((1, 1, BN), lambda j: (j, 0, 0))],
        out_specs=pl.BlockSpec((G, 1), lambda j: (0, 0)),
        out_shape=jax.ShapeDtypeStruct((G, 1), jnp.float32),
    )(p0, p1, h, x, wihT, whhT, bih2, bhh2, wxr, whr, bh2, bidx3)


def kernel(x, edge_index, batch_index, weight, w_ih, w_hh, b_ih, b_hh,
           w_head, b_head):
    src = edge_index[0]
    dst = edge_index[1]
    pad = EPAD - E
    src_p = jnp.concatenate([src, jnp.zeros((pad,), jnp.int32)])
    dst_p = jnp.concatenate([dst, jnp.full((pad,), N, jnp.int32)])
    epack = jnp.stack([src_p, dst_p], 0).reshape(2, NW * CPW, CHUNK)
    epack = epack.transpose(1, 0, 2)
    zeros_rows = jnp.zeros((NPAD, H), jnp.float32)
    wihT = w_ih.T
    whhT = w_hh.T
    bih2 = b_ih.reshape(1, 3 * H)
    bhh2 = b_hh.reshape(1, 3 * H)
    wxr = w_head[:, :H]
    whr = w_head[:, H:]
    bh2 = b_head.reshape(1, 1)
    bidx3 = batch_index.reshape(NB, 1, BN)

    h = x
    m = _matmul(x, weight[0])
    for i in range(L - 1):
        p = _sc_edge_aggregate(m, epack, zeros_rows)
        h, m = _gru_next(p[0, :N], p[1, :N], h, wihT, whhT, bih2, bhh2,
                         weight[i + 1])
    p = _sc_edge_aggregate(m, epack, zeros_rows)
    out = _final(p[0, :N], p[1, :N], h, x, wihT, whhT, bih2, bhh2, wxr, whr,
                 bh2, bidx3)
    return out[:, 0]


# speedup vs baseline: 2.7112x; 1.1381x over previous
import functools

import jax
import jax.numpy as jnp
from jax import lax
from jax.experimental import pallas as pl
from jax.experimental.pallas import tpu as pltpu
from jax.experimental.pallas import tpu_sc as plsc

N = 10000
E = 320000
H = 128
G = 64
L = 3

NC, NS = 2, 16
NW = NC * NS
CHUNK = 128
EPW = 10240
CPW = EPW // CHUNK
HCPW = CPW // 2
EPAD = EPW * NW
NPAD = 10112
ZR = NPAD // NS

BN = 1000
NB = N // BN

_PREC = lax.Precision.HIGHEST

_sc_mesh = plsc.VectorSubcoreMesh(core_axis_name="c", subcore_axis_name="s")


@functools.partial(
    pl.kernel,
    out_type=jax.ShapeDtypeStruct((NC, NPAD, H), jnp.float32),
    mesh=_sc_mesh,
    scratch_types=[
        pltpu.VMEM((HCPW, CHUNK), jnp.int32),
        pltpu.VMEM((HCPW, CHUNK), jnp.int32),
        pltpu.VMEM((2, CHUNK, H), jnp.float32),
        pltpu.VMEM_SHARED((NPAD, H), jnp.float32),
        pltpu.SemaphoreType.DMA((2,)),
    ],
)
def _sc_edge_aggregate(m_hbm, srcs_hbm, dsts_hbm, zeros_hbm, out_hbm,
                       srcv, dstv, rows, agg_sh, gsem):
    c = lax.axis_index("c")
    s = lax.axis_index("s")
    pltpu.sync_copy(zeros_hbm.at[pl.ds(s * ZR, ZR)], agg_sh.at[pl.ds(s * ZR, ZR)])
    wid = c * NS + s
    base = wid * CPW
    pltpu.sync_copy(srcs_hbm.at[pl.ds(base, HCPW)], srcv)
    pltpu.sync_copy(dsts_hbm.at[pl.ds(base, HCPW)], dstv)
    plsc.subcore_barrier()

    pltpu.async_copy(m_hbm.at[srcv.at[0]], rows.at[0], gsem.at[0])

    def body(g, carry):
        buf = lax.rem(g, 2)
        row = lax.rem(g, HCPW)
        pltpu.make_async_copy(m_hbm.at[srcv.at[row]], rows.at[buf],
                              gsem.at[buf]).wait()

        @pl.when(g == HCPW - 1)
        def _refresh_src():
            pltpu.sync_copy(srcs_hbm.at[pl.ds(base + HCPW, HCPW)], srcv)

        @pl.when(g + 1 < CPW)
        def _prefetch():
            nb = lax.rem(g + 1, 2)
            nrow = lax.rem(g + 1, HCPW)
            pltpu.async_copy(m_hbm.at[srcv.at[nrow]], rows.at[nb],
                             gsem.at[nb])

        @pl.when(g == HCPW)
        def _refresh_dst():
            pltpu.sync_copy(dsts_hbm.at[pl.ds(base + HCPW, HCPW)], dstv)

        pltpu.sync_copy(rows.at[buf], agg_sh.at[dstv.at[row]], add=True)
        return carry

    lax.fori_loop(0, CPW, body, 0)
    plsc.subcore_barrier()
    pltpu.sync_copy(agg_sh.at[pl.ds(s * ZR, ZR)],
                    out_hbm.at[c, pl.ds(s * ZR, ZR)])


def _sigmoid(v):
    return 1.0 / (1.0 + jnp.exp(-v))


def _mm_body(x_ref, w_ref, o_ref):
    o_ref[...] = jnp.dot(x_ref[...], w_ref[...], precision=_PREC,
                         preferred_element_type=jnp.float32)


def _matmul(x, w):
    return pl.pallas_call(
        _mm_body,
        grid=(NB,),
        in_specs=[pl.BlockSpec((BN, H), lambda j: (j, 0)),
                  pl.BlockSpec((H, H), lambda j: (0, 0))],
        out_specs=pl.BlockSpec((BN, H), lambda j: (j, 0)),
        out_shape=jax.ShapeDtypeStruct((N, H), jnp.float32),
    )(x, w)


def _gru_math(p0, p1, h, wih, whh, bih, bhh):
    agg = p0[...] + p1[...]
    gi = jnp.dot(agg, wih[...], precision=_PREC,
                 preferred_element_type=jnp.float32) + bih[...]
    gh = jnp.dot(h[...], whh[...], precision=_PREC,
                 preferred_element_type=jnp.float32) + bhh[...]
    hv = h[...]
    r = _sigmoid(gi[:, 0:H] + gh[:, 0:H])
    z = _sigmoid(gi[:, H:2 * H] + gh[:, H:2 * H])
    n = jnp.tanh(gi[:, 2 * H:3 * H] + r * gh[:, 2 * H:3 * H])
    return (1.0 - z) * n + z * hv


def _gru_body(p0, p1, h, wih, whh, bih, bhh, wn, ho, mo):
    hn = _gru_math(p0, p1, h, wih, whh, bih, bhh)
    ho[...] = hn
    mo[...] = jnp.dot(hn, wn[...], precision=_PREC,
                      preferred_element_type=jnp.float32)


def _gru_next(p0, p1, h, wihT, whhT, bih2, bhh2, wnext):
    full = lambda r, c: pl.BlockSpec((r, c), lambda j: (0, 0))
    blk = pl.BlockSpec((BN, H), lambda j: (j, 0))
    return pl.pallas_call(
        _gru_body,
        grid=(NB,),
        in_specs=[blk, blk, blk, full(H, 3 * H), full(H, 3 * H),
                  full(1, 3 * H), full(1, 3 * H), full(H, H)],
        out_specs=[blk, blk],
        out_shape=[jax.ShapeDtypeStruct((N, H), jnp.float32),
                   jax.ShapeDtypeStruct((N, H), jnp.float32)],
    )(p0, p1, h, wihT, whhT, bih2, bhh2, wnext)


def _final_body(p0, p1, h, x, wih, whh, bih, bhh, wxr, whr, bh, bidx, out):
    j = pl.program_id(0)
    hn = _gru_math(p0, p1, h, wih, whh, bih, bhh)
    lg2 = x[...] * wxr[...] + hn * whr[...]
    rowlog = jnp.sum(lg2, axis=1, keepdims=True) + bh[...]
    bi = bidx[0]
    onehot = (lax.broadcasted_iota(jnp.int32, (G, BN), 0) == bi
              ).astype(jnp.float32)
    pp = jnp.dot(onehot, rowlog, precision=_PREC,
                 preferred_element_type=jnp.float32)

    @pl.when(j == 0)
    def _init():
        out[...] = jnp.zeros_like(out)

    out[...] += pp

    @pl.when(j == NB - 1)
    def _fin():
        out[...] = _sigmoid(out[...])


def _final(p0, p1, h, x, wihT, whhT, bih2, bhh2, wxr, whr, bh2, bidx3):
    full = lambda r, c: pl.BlockSpec((r, c), lambda j: (0, 0))
    blk = pl.BlockSpec((BN, H), lambda j: (j, 0))
    return pl.pallas_call(
        _final_body,
        grid=(NB,),
        in_specs=[blk, blk, blk, blk, full(H, 3 * H), full(H, 3 * H),
                  full(1, 3 * H), full(1, 3 * H), full(1, H), full(1, H),
                  full(1, 1), pl.BlockSpec((1, 1, BN), lambda j: (j, 0, 0))],
        out_specs=pl.BlockSpec((G, 1), lambda j: (0, 0)),
        out_shape=jax.ShapeDtypeStruct((G, 1), jnp.float32),
    )(p0, p1, h, x, wihT, whhT, bih2, bhh2, wxr, whr, bh2, bidx3)


def kernel(x, edge_index, batch_index, weight, w_ih, w_hh, b_ih, b_hh,
           w_head, b_head):
    src = edge_index[0]
    dst = edge_index[1]
    pad = EPAD - E
    src_p = jnp.concatenate([src, jnp.zeros((pad,), jnp.int32)])
    dst_p = jnp.concatenate([dst, jnp.full((pad,), N, jnp.int32)])
    srcs = src_p.reshape(NW * CPW, CHUNK)
    dsts = dst_p.reshape(NW * CPW, CHUNK)
    zeros_rows = jnp.zeros((NPAD, H), jnp.float32)
    wihT = w_ih.T
    whhT = w_hh.T
    bih2 = b_ih.reshape(1, 3 * H)
    bhh2 = b_hh.reshape(1, 3 * H)
    wxr = w_head[:, :H]
    whr = w_head[:, H:]
    bh2 = b_head.reshape(1, 1)
    bidx3 = batch_index.reshape(NB, 1, BN)

    h = x
    m = _matmul(x, weight[0])
    for i in range(L - 1):
        p = _sc_edge_aggregate(m, srcs, dsts, zeros_rows)
        h, m = _gru_next(p[0, :N], p[1, :N], h, wihT, whhT, bih2, bhh2,
                         weight[i + 1])
    p = _sc_edge_aggregate(m, srcs, dsts, zeros_rows)
    out = _final(p[0, :N], p[1, :N], h, x, wihT, whhT, bih2, bhh2, wxr, whr,
                 bh2, bidx3)
    return out[:, 0]


# speedup vs baseline: 3.0413x; 1.1218x over previous
import functools

import jax
import jax.numpy as jnp
from jax import lax
from jax.experimental import pallas as pl
from jax.experimental.pallas import tpu as pltpu
from jax.experimental.pallas import tpu_sc as plsc

N = 10000
E = 320000
H = 128
G = 64
L = 3

NC, NS = 2, 16
NW = NC * NS
CHUNK = 64
CPW = 160
EPW = CPW * CHUNK
NBUF = 4
DSTAGE = 40
EPAD = EPW * NW
NPAD = 10112
ZR = NPAD // NS

BN = 1000
NB = N // BN

_PREC = lax.Precision.HIGHEST

_sc_mesh = plsc.VectorSubcoreMesh(core_axis_name="c", subcore_axis_name="s")


@functools.partial(
    pl.kernel,
    out_type=jax.ShapeDtypeStruct((NC, NPAD, H), jnp.float32),
    mesh=_sc_mesh,
    scratch_types=[
        pltpu.VMEM((EPW,), jnp.int32),
        pltpu.VMEM((DSTAGE, CHUNK), jnp.int32),
        pltpu.VMEM((NBUF, CHUNK, H), jnp.float32),
        pltpu.VMEM_SHARED((NPAD, H), jnp.float32),
        pltpu.SemaphoreType.DMA((NBUF,)),
    ],
)
def _sc_edge_aggregate(m_hbm, srcs_hbm, dsts_hbm, zeros_hbm, out_hbm,
                       srcv, dstv, rows, agg_sh, gsem):
    c = lax.axis_index("c")
    s = lax.axis_index("s")
    pltpu.sync_copy(zeros_hbm.at[pl.ds(s * ZR, ZR)], agg_sh.at[pl.ds(s * ZR, ZR)])
    wid = c * NS + s
    pltpu.sync_copy(srcs_hbm.at[pl.ds(wid * EPW, EPW)], srcv)
    dbase = wid * CPW
    pltpu.sync_copy(dsts_hbm.at[pl.ds(dbase, DSTAGE)], dstv)
    plsc.subcore_barrier()

    for b in range(NBUF - 1):
        pltpu.async_copy(m_hbm.at[srcv.at[pl.ds(b * CHUNK, CHUNK)]],
                         rows.at[b], gsem.at[b])

    def body(g, carry):
        buf = lax.rem(g, NBUF)
        pltpu.make_async_copy(m_hbm.at[srcv.at[pl.ds(g * CHUNK, CHUNK)]],
                              rows.at[buf], gsem.at[buf]).wait()

        @pl.when(g + NBUF - 1 < CPW)
        def _prefetch():
            nb = lax.rem(g + NBUF - 1, NBUF)
            pltpu.async_copy(
                m_hbm.at[srcv.at[pl.ds((g + NBUF - 1) * CHUNK, CHUNK)]],
                rows.at[nb], gsem.at[nb])

        drow = lax.rem(g, DSTAGE)

        @pl.when((drow == 0) & (g > 0))
        def _refresh_dst():
            start = pl.multiple_of(dbase + g, 8)
            pltpu.sync_copy(dsts_hbm.at[pl.ds(start, DSTAGE)], dstv)

        pltpu.sync_copy(rows.at[buf], agg_sh.at[dstv.at[drow]], add=True)
        return carry

    lax.fori_loop(0, CPW, body, 0)
    plsc.subcore_barrier()
    pltpu.sync_copy(agg_sh.at[pl.ds(s * ZR, ZR)],
                    out_hbm.at[c, pl.ds(s * ZR, ZR)])


def _sigmoid(v):
    return 1.0 / (1.0 + jnp.exp(-v))


def _mm_body(x_ref, w_ref, o_ref):
    o_ref[...] = jnp.dot(x_ref[...], w_ref[...], precision=_PREC,
                         preferred_element_type=jnp.float32)


def _matmul(x, w):
    return pl.pallas_call(
        _mm_body,
        grid=(NB,),
        in_specs=[pl.BlockSpec((BN, H), lambda j: (j, 0)),
                  pl.BlockSpec((H, H), lambda j: (0, 0))],
        out_specs=pl.BlockSpec((BN, H), lambda j: (j, 0)),
        out_shape=jax.ShapeDtypeStruct((N, H), jnp.float32),
    )(x, w)


def _gru_math(p0, p1, h, wih, whh, bih, bhh):
    agg = p0[...] + p1[...]
    gi = jnp.dot(agg, wih[...], precision=_PREC,
                 preferred_element_type=jnp.float32) + bih[...]
    gh = jnp.dot(h[...], whh[...], precision=_PREC,
                 preferred_element_type=jnp.float32) + bhh[...]
    hv = h[...]
    r = _sigmoid(gi[:, 0:H] + gh[:, 0:H])
    z = _sigmoid(gi[:, H:2 * H] + gh[:, H:2 * H])
    n = jnp.tanh(gi[:, 2 * H:3 * H] + r * gh[:, 2 * H:3 * H])
    return (1.0 - z) * n + z * hv


def _gru_body(p0, p1, h, wih, whh, bih, bhh, wn, ho, mo):
    hn = _gru_math(p0, p1, h, wih, whh, bih, bhh)
    ho[...] = hn
    mo[...] = jnp.dot(hn, wn[...], precision=_PREC,
                      preferred_element_type=jnp.float32)


def _gru_next(p0, p1, h, wihT, whhT, bih2, bhh2, wnext):
    full = lambda r, c: pl.BlockSpec((r, c), lambda j: (0, 0))
    blk = pl.BlockSpec((BN, H), lambda j: (j, 0))
    return pl.pallas_call(
        _gru_body,
        grid=(NB,),
        in_specs=[blk, blk, blk, full(H, 3 * H), full(H, 3 * H),
                  full(1, 3 * H), full(1, 3 * H), full(H, H)],
        out_specs=[blk, blk],
        out_shape=[jax.ShapeDtypeStruct((N, H), jnp.float32),
                   jax.ShapeDtypeStruct((N, H), jnp.float32)],
    )(p0, p1, h, wihT, whhT, bih2, bhh2, wnext)


def _final_body(p0, p1, h, x, wih, whh, bih, bhh, wxr, whr, bh, bidx, out):
    j = pl.program_id(0)
    hn = _gru_math(p0, p1, h, wih, whh, bih, bhh)
    lg2 = x[...] * wxr[...] + hn * whr[...]
    rowlog = jnp.sum(lg2, axis=1, keepdims=True) + bh[...]
    bi = bidx[0]
    onehot = (lax.broadcasted_iota(jnp.int32, (G, BN), 0) == bi
              ).astype(jnp.float32)
    pp = jnp.dot(onehot, rowlog, precision=_PREC,
                 preferred_element_type=jnp.float32)

    @pl.when(j == 0)
    def _init():
        out[...] = jnp.zeros_like(out)

    out[...] += pp

    @pl.when(j == NB - 1)
    def _fin():
        out[...] = _sigmoid(out[...])


def _final(p0, p1, h, x, wihT, whhT, bih2, bhh2, wxr, whr, bh2, bidx3):
    full = lambda r, c: pl.BlockSpec((r, c), lambda j: (0, 0))
    blk = pl.BlockSpec((BN, H), lambda j: (j, 0))
    return pl.pallas_call(
        _final_body,
        grid=(NB,),
        in_specs=[blk, blk, blk, blk, full(H, 3 * H), full(H, 3 * H),
                  full(1, 3 * H), full(1, 3 * H), full(1, H), full(1, H),
                  full(1, 1), pl.BlockSpec((1, 1, BN), lambda j: (j, 0, 0))],
        out_specs=pl.BlockSpec((G, 1), lambda j: (0, 0)),
        out_shape=jax.ShapeDtypeStruct((G, 1), jnp.float32),
    )(p0, p1, h, x, wihT, whhT, bih2, bhh2, wxr, whr, bh2, bidx3)


def kernel(x, edge_index, batch_index, weight, w_ih, w_hh, b_ih, b_hh,
           w_head, b_head):
    src = edge_index[0]
    dst = edge_index[1]
    pad = EPAD - E
    srcs = jnp.concatenate([src, jnp.zeros((pad,), jnp.int32)])
    dst_p = jnp.concatenate([dst, jnp.full((pad,), N, jnp.int32)])
    dsts = dst_p.reshape(NW * CPW, CHUNK)
    zeros_rows = jnp.zeros((NPAD, H), jnp.float32)
    wihT = w_ih.T
    whhT = w_hh.T
    bih2 = b_ih.reshape(1, 3 * H)
    bhh2 = b_hh.reshape(1, 3 * H)
    wxr = w_head[:, :H]
    whr = w_head[:, H:]
    bh2 = b_head.reshape(1, 1)
    bidx3 = batch_index.reshape(NB, 1, BN)

    h = x
    m = _matmul(x, weight[0])
    for i in range(L - 1):
        p = _sc_edge_aggregate(m, srcs, dsts, zeros_rows)
        h, m = _gru_next(p[0, :N], p[1, :N], h, wihT, whhT, bih2, bhh2,
                         weight[i + 1])
    p = _sc_edge_aggregate(m, srcs, dsts, zeros_rows)
    out = _final(p[0, :N], p[1, :N], h, x, wihT, whhT, bih2, bhh2, wxr, whr,
                 bh2, bidx3)
    return out[:, 0]
